# baseline (speedup 1.0000x reference)
import functools

import jax
import jax.numpy as jnp
from jax import lax
from jax.experimental import pallas as pl
from jax.experimental.pallas import tpu as pltpu
from jax.experimental.pallas import tpu_sc as plsc

MAX_POSITIONS = 8192
HIDDEN_SIZE = 1024

NUM_CORES = 2
NUM_SUBCORES = 16
NUM_WORKERS = NUM_CORES * NUM_SUBCORES
ROWS_PER_WORKER = MAX_POSITIONS // NUM_WORKERS


def _copy_body(table_hbm, out_hbm):
    wid = lax.axis_index("s") * NUM_CORES + lax.axis_index("c")
    base = wid * ROWS_PER_WORKER
    pltpu.sync_copy(
        table_hbm.at[pl.ds(base, ROWS_PER_WORKER)],
        out_hbm.at[pl.ds(base, ROWS_PER_WORKER)],
    )


@jax.jit
def _sc_copy(table):
    mesh = plsc.VectorSubcoreMesh(core_axis_name="c", subcore_axis_name="s")
    return pl.kernel(
        _copy_body,
        mesh=mesh,
        out_type=jax.ShapeDtypeStruct((MAX_POSITIONS, HIDDEN_SIZE), jnp.float32),
    )(table)


def kernel(inputs, table):
    del inputs
    return _sc_copy(table)


# speedup vs baseline: 23.1729x; 23.1729x over previous
import functools

import jax
import jax.numpy as jnp
from jax import lax
from jax.experimental import pallas as pl
from jax.experimental.pallas import tpu as pltpu
from jax.experimental.pallas import tpu_sc as plsc

MAX_POSITIONS = 8192
HIDDEN_SIZE = 1024

NUM_CORES = 2
NUM_SUBCORES = 16
NUM_WORKERS = NUM_CORES * NUM_SUBCORES
ROWS_PER_WORKER = MAX_POSITIONS // NUM_WORKERS


CHUNK_ROWS = 32
NUM_CHUNKS = ROWS_PER_WORKER // CHUNK_ROWS


def _copy_body(table_hbm, out_hbm, buf0, buf1, si0, si1, so0, so1):
    wid = lax.axis_index("s") * NUM_CORES + lax.axis_index("c")
    base = wid * ROWS_PER_WORKER
    bufs = (buf0, buf1)
    sin = (si0, si1)
    sout = (so0, so1)

    in_cp = [None] * NUM_CHUNKS
    out_cp = [None] * NUM_CHUNKS
    in_cp[0] = pltpu.async_copy(
        table_hbm.at[pl.ds(base, CHUNK_ROWS)], bufs[0], sin[0]
    )
    for i in range(NUM_CHUNKS):
        j = i % 2
        in_cp[i].wait()
        if i + 1 < NUM_CHUNKS:
            jn = (i + 1) % 2
            if i >= 1:
                out_cp[i - 1].wait()
            in_cp[i + 1] = pltpu.async_copy(
                table_hbm.at[pl.ds(base + (i + 1) * CHUNK_ROWS, CHUNK_ROWS)],
                bufs[jn],
                sin[jn],
            )
        out_cp[i] = pltpu.async_copy(
            bufs[j], out_hbm.at[pl.ds(base + i * CHUNK_ROWS, CHUNK_ROWS)], sout[j]
        )
    out_cp[NUM_CHUNKS - 2].wait()
    out_cp[NUM_CHUNKS - 1].wait()


@jax.jit
def _sc_copy(table):
    mesh = plsc.VectorSubcoreMesh(core_axis_name="c", subcore_axis_name="s")
    return pl.kernel(
        _copy_body,
        mesh=mesh,
        out_type=jax.ShapeDtypeStruct((MAX_POSITIONS, HIDDEN_SIZE), jnp.float32),
        scratch_types=[
            pltpu.VMEM((CHUNK_ROWS, HIDDEN_SIZE), jnp.float32),
            pltpu.VMEM((CHUNK_ROWS, HIDDEN_SIZE), jnp.float32),
            pltpu.SemaphoreType.DMA,
            pltpu.SemaphoreType.DMA,
            pltpu.SemaphoreType.DMA,
            pltpu.SemaphoreType.DMA,
        ],
    )(table)


def kernel(inputs, table):
    del inputs
    return _sc_copy(table)


# speedup vs baseline: 24.2881x; 1.0481x over previous
import functools

import jax
import jax.numpy as jnp
from jax import lax
from jax.experimental import pallas as pl
from jax.experimental.pallas import tpu as pltpu
from jax.experimental.pallas import tpu_sc as plsc

MAX_POSITIONS = 8192
HIDDEN_SIZE = 1024

NUM_CORES = 2
NUM_SUBCORES = 16
NUM_WORKERS = NUM_CORES * NUM_SUBCORES
ROWS_PER_WORKER = MAX_POSITIONS // NUM_WORKERS


CHUNK_ROWS = 16
NUM_CHUNKS = ROWS_PER_WORKER // CHUNK_ROWS
NUM_BUFS = 4


def _copy_body(table_hbm, out_hbm, *scratch):
    bufs = scratch[:NUM_BUFS]
    sin = scratch[NUM_BUFS : 2 * NUM_BUFS]
    sout = scratch[2 * NUM_BUFS :]
    wid = lax.axis_index("s") * NUM_CORES + lax.axis_index("c")
    base = wid * ROWS_PER_WORKER

    in_cp = [None] * NUM_CHUNKS
    out_cp = [None] * NUM_CHUNKS
    for i in range(min(NUM_BUFS, NUM_CHUNKS)):
        in_cp[i] = pltpu.async_copy(
            table_hbm.at[pl.ds(base + i * CHUNK_ROWS, CHUNK_ROWS)],
            bufs[i],
            sin[i],
        )
    for i in range(NUM_CHUNKS):
        b = i % NUM_BUFS
        in_cp[i].wait()
        out_cp[i] = pltpu.async_copy(
            bufs[b], out_hbm.at[pl.ds(base + i * CHUNK_ROWS, CHUNK_ROWS)], sout[b]
        )
        j = i + NUM_BUFS
        if j < NUM_CHUNKS:
            out_cp[i].wait()
            in_cp[j] = pltpu.async_copy(
                table_hbm.at[pl.ds(base + j * CHUNK_ROWS, CHUNK_ROWS)],
                bufs[b],
                sin[b],
            )
    for i in range(max(0, NUM_CHUNKS - NUM_BUFS), NUM_CHUNKS):
        out_cp[i].wait()


@jax.jit
def _sc_copy(table):
    mesh = plsc.VectorSubcoreMesh(core_axis_name="c", subcore_axis_name="s")
    return pl.kernel(
        _copy_body,
        mesh=mesh,
        out_type=jax.ShapeDtypeStruct((MAX_POSITIONS, HIDDEN_SIZE), jnp.float32),
        scratch_types=(
            [pltpu.VMEM((CHUNK_ROWS, HIDDEN_SIZE), jnp.float32)] * NUM_BUFS
            + [pltpu.SemaphoreType.DMA] * (2 * NUM_BUFS)
        ),
    )(table)


def kernel(inputs, table):
    del inputs
    return _sc_copy(table)
